# baseline (speedup 1.0000x reference)
import functools

import jax
import jax.numpy as jnp
from jax import lax
from jax.experimental import pallas as pl
from jax.experimental.pallas import tpu as pltpu
from jax.experimental.pallas import tpu_sc as plsc

_NUM_CORES = 2
_NUM_SUBCORES = 16
_LANES = 16
_NBUF = 4
_C = 128
_NSPLIT = 4


def kernel(x, token_table, pos_table):
    B, S = x.shape
    V, D = token_table.shape
    n_tok = B * S
    n_chunks = n_tok // _C
    nw = _NUM_CORES * _NUM_SUBCORES
    chunks_per_w = n_chunks // nw
    n_steps = chunks_per_w // _NBUF

    idx = x.reshape(n_chunks, _C).astype(jnp.int32)

    mesh = plsc.VectorSubcoreMesh(core_axis_name="c", subcore_axis_name="s")

    @functools.partial(
        pl.kernel,
        mesh=mesh,
        out_type=jax.ShapeDtypeStruct((n_tok, D), jnp.float32),
        scratch_types=[
            pltpu.VMEM((chunks_per_w, _C), jnp.int32),
            pltpu.VMEM((S, D), jnp.float32),
            [pltpu.VMEM((_C, D), jnp.float32)] * _NBUF,
            [pltpu.SemaphoreType.DMA] * (_NSPLIT * _NBUF),
            [pltpu.SemaphoreType.DMA] * _NBUF,
        ],
    )
    def emb_kernel(idx_hbm, tok_hbm, pos_hbm, out_hbm, idx_v, pos_v, bufs,
                   gsems, ssems):
        wid = lax.axis_index("s") * _NUM_CORES + lax.axis_index("c")
        base = wid * chunks_per_w
        pltpu.sync_copy(pos_hbm, pos_v)
        pltpu.sync_copy(idx_hbm.at[pl.ds(base, chunks_per_w)], idx_v)

        H = _C // _NSPLIT

        def gather_part(kk, b, h):
            return pltpu.make_async_copy(
                tok_hbm.at[idx_v.at[kk, pl.ds(h * H, H)]],
                bufs[b].at[pl.ds(h * H, H)],
                gsems[_NSPLIT * b + h])

        def gather_start(kk, b):
            for h in range(_NSPLIT):
                gather_part(kk, b, h).start()

        def gather_wait(kk, b):
            for h in range(_NSPLIT):
                gather_part(kk, b, h).wait()

        def store(kk, b):
            return pltpu.make_async_copy(
                bufs[b], out_hbm.at[pl.ds((base + kk) * _C, _C)], ssems[b])

        gather_start(0, 0)
        gather_start(1, 1)

        def step_body(k, carry):
            for b in range(_NBUF):
                kk = k * _NBUF + b
                gather_wait(kk, b)

                b2 = (b + 2) % _NBUF

                @pl.when(kk >= 2)
                def _wait_prev_store():
                    store(kk - 2, b2).wait()

                @pl.when(kk + 2 < chunks_per_w)
                def _issue_next_gather():
                    gather_start(kk + 2, b2)

                start = ((base + kk) * _C) % S
                buf = bufs[b]

                @plsc.parallel_loop(0, _C, step=1, unroll=4)
                def row_add(i):
                    r = start + i
                    r = r - jnp.where(r >= S, S, 0)
                    vals = [
                        buf[i, pl.ds(j * _LANES, _LANES)]
                        + pos_v[r, pl.ds(j * _LANES, _LANES)]
                        for j in range(D // _LANES)
                    ]
                    for j in range(D // _LANES):
                        buf[i, pl.ds(j * _LANES, _LANES)] = vals[j]

                store(kk, b).start()
            return carry

        lax.fori_loop(0, n_steps, step_body, 0)

        store(chunks_per_w - 2, (chunks_per_w - 2) % _NBUF).wait()
        store(chunks_per_w - 1, (chunks_per_w - 1) % _NBUF).wait()

    out = emb_kernel(idx, token_table, pos_table)
    return out.reshape(B, S, D)


# speedup vs baseline: 1.0024x; 1.0024x over previous
import functools

import jax
import jax.numpy as jnp
from jax import lax
from jax.experimental import pallas as pl
from jax.experimental.pallas import tpu as pltpu
from jax.experimental.pallas import tpu_sc as plsc

_NUM_CORES = 2
_NUM_SUBCORES = 16
_LANES = 16
_NBUF = 4
_C = 128
_NSPLIT = 2


def kernel(x, token_table, pos_table):
    B, S = x.shape
    V, D = token_table.shape
    n_tok = B * S
    n_chunks = n_tok // _C
    nw = _NUM_CORES * _NUM_SUBCORES
    chunks_per_w = n_chunks // nw
    n_steps = chunks_per_w // _NBUF

    idx = x.reshape(n_chunks, _C).astype(jnp.int32)

    mesh = plsc.VectorSubcoreMesh(core_axis_name="c", subcore_axis_name="s")

    @functools.partial(
        pl.kernel,
        mesh=mesh,
        out_type=jax.ShapeDtypeStruct((n_tok, D), jnp.float32),
        scratch_types=[
            pltpu.VMEM((chunks_per_w, _C), jnp.int32),
            pltpu.VMEM((S, D), jnp.float32),
            [pltpu.VMEM((_C, D), jnp.float32)] * _NBUF,
            [pltpu.SemaphoreType.DMA] * (_NSPLIT * _NBUF),
            [pltpu.SemaphoreType.DMA] * _NBUF,
        ],
    )
    def emb_kernel(idx_hbm, tok_hbm, pos_hbm, out_hbm, idx_v, pos_v, bufs,
                   gsems, ssems):
        wid = lax.axis_index("s") * _NUM_CORES + lax.axis_index("c")
        base = wid * chunks_per_w
        pltpu.sync_copy(pos_hbm, pos_v)
        pltpu.sync_copy(idx_hbm.at[pl.ds(base, chunks_per_w)], idx_v)

        H = _C // _NSPLIT

        def gather_part(kk, b, h):
            return pltpu.make_async_copy(
                tok_hbm.at[idx_v.at[kk, pl.ds(h * H, H)]],
                bufs[b].at[pl.ds(h * H, H)],
                gsems[_NSPLIT * b + h])

        def gather_start(kk, b):
            for h in range(_NSPLIT):
                gather_part(kk, b, h).start()

        def gather_wait(kk, b):
            for h in range(_NSPLIT):
                gather_part(kk, b, h).wait()

        def store(kk, b):
            return pltpu.make_async_copy(
                bufs[b], out_hbm.at[pl.ds((base + kk) * _C, _C)], ssems[b])

        gather_start(0, 0)
        gather_start(1, 1)

        def step_body(k, carry):
            for b in range(_NBUF):
                kk = k * _NBUF + b
                gather_wait(kk, b)

                b2 = (b + 2) % _NBUF

                @pl.when(kk >= 2)
                def _wait_prev_store():
                    store(kk - 2, b2).wait()

                @pl.when(kk + 2 < chunks_per_w)
                def _issue_next_gather():
                    gather_start(kk + 2, b2)

                start = ((base + kk) * _C) % S
                buf = bufs[b]

                @plsc.parallel_loop(0, _C, step=1, unroll=4)
                def row_add(i):
                    r = start + i
                    r = r - jnp.where(r >= S, S, 0)
                    vals = [
                        buf[i, pl.ds(j * _LANES, _LANES)]
                        + pos_v[r, pl.ds(j * _LANES, _LANES)]
                        for j in range(D // _LANES)
                    ]
                    for j in range(D // _LANES):
                        buf[i, pl.ds(j * _LANES, _LANES)] = vals[j]

                store(kk, b).start()
            return carry

        lax.fori_loop(0, n_steps, step_body, 0)

        store(chunks_per_w - 2, (chunks_per_w - 2) % _NBUF).wait()
        store(chunks_per_w - 1, (chunks_per_w - 1) % _NBUF).wait()

    out = emb_kernel(idx, token_table, pos_table)
    return out.reshape(B, S, D)
